# Initial kernel scaffold; baseline (speedup 1.0000x reference)
import functools

import jax
import jax.numpy as jnp
from jax import lax
from jax.experimental import pallas as pl
from jax.experimental.pallas import tpu as pltpu
from jax.experimental.pallas import tpu_sc as plsc

_DIM = 64
_VOCAB = 100000
_T1_ROWS = 1792
_T2_ROWS = 14336
_T3_ROWS = 83616
_W3_BLK = 6432
_BR = 256
_MV1, _MV2, _MV3 = 127.0, 31.0, 7.0


def _scales_body(w1, w2, w3, s1, s2, s3, out, acc):
    i = pl.program_id(0)

    @pl.when(i == 0)
    def _():
        acc[0:1, 0:_DIM] = jnp.zeros((1, _DIM), jnp.float32)
        acc[1:2, 0:_DIM] = jnp.max(jnp.abs(w1[...] * s1[...]), axis=0)[None, :]
        acc[2:3, 0:_DIM] = jnp.max(jnp.abs(w2[...] * s2[...]), axis=0)[None, :]

    m3 = jnp.max(jnp.abs(w3[...] * s3[...]), axis=0)[None, :]
    acc[0:1, 0:_DIM] = jnp.maximum(acc[0:1, 0:_DIM], m3)

    @pl.when(i == pl.num_programs(0) - 1)
    def _():
        sc1 = jnp.clip(jnp.max(acc[1:2, 0:_DIM]), 1e-8, None) / _MV1
        sc2 = jnp.clip(jnp.max(acc[2:3, 0:_DIM]), 1e-8, None) / _MV2
        sc3 = jnp.clip(jnp.max(acc[0:1, 0:_DIM]), 1e-8, None) / _MV3
        r = lax.broadcasted_iota(jnp.int32, (8, 128), 0)
        out[...] = jnp.where(r == 0, sc1, jnp.where(r == 1, sc2, sc3))


def _compute_scales(w1, w2, w3, s1, s2, s3):
    return pl.pallas_call(
        _scales_body,
        grid=(_T3_ROWS // _W3_BLK,),
        in_specs=[
            pl.BlockSpec((_T1_ROWS, _DIM), lambda i: (0, 0)),
            pl.BlockSpec((_T2_ROWS, _DIM), lambda i: (0, 0)),
            pl.BlockSpec((_W3_BLK, _DIM), lambda i: (i, 0)),
            pl.BlockSpec((1, _DIM), lambda i: (0, 0)),
            pl.BlockSpec((1, _DIM), lambda i: (0, 0)),
            pl.BlockSpec((1, _DIM), lambda i: (0, 0)),
        ],
        out_specs=pl.BlockSpec((8, 128), lambda i: (0, 0)),
        out_shape=jax.ShapeDtypeStruct((8, 128), jnp.float32),
        scratch_shapes=[pltpu.VMEM((8, 128), jnp.float32)],
    )(w1, w2, w3, s1, s2, s3)


def _quant_body(w0, w1, w2, w3, s0, s1, s2, s3, scales, out):
    i = pl.program_id(0)
    sc1 = scales[0, 0]
    sc2 = scales[1, 0]
    sc3 = scales[2, 0]

    def q(x, sc, mv):
        return jnp.clip(jnp.round(x / sc), -mv, mv) * sc

    x0 = w0[...] * s0[...]
    x1 = q(w1[...] * s1[...], sc1, _MV1)
    x2 = q(w2[...] * s2[...], sc2, _MV2)
    x3 = q(w3[...] * s3[...], sc3, _MV3)
    out[...] = jnp.where(i == 0, x0,
               jnp.where(i <= 7, x1,
               jnp.where(i <= 63, x2, x3)))


def _build_table(w0, w1, w2, w3, s0, s1, s2, s3, scales):
    n_blocks = pl.cdiv(_VOCAB, _BR)
    return pl.pallas_call(
        _quant_body,
        grid=(n_blocks,),
        in_specs=[
            pl.BlockSpec((_BR, _DIM), lambda i: (0, 0)),
            pl.BlockSpec((_BR, _DIM), lambda i: (jnp.clip(i - 1, 0, 6), 0)),
            pl.BlockSpec((_BR, _DIM), lambda i: (jnp.clip(i - 8, 0, 55), 0)),
            pl.BlockSpec((_BR, _DIM), lambda i: (jnp.clip(i - 64, 0, 326), 0)),
            pl.BlockSpec((1, _DIM), lambda i: (0, 0)),
            pl.BlockSpec((1, _DIM), lambda i: (0, 0)),
            pl.BlockSpec((1, _DIM), lambda i: (0, 0)),
            pl.BlockSpec((1, _DIM), lambda i: (0, 0)),
            pl.BlockSpec(memory_space=pltpu.SMEM),
        ],
        out_specs=pl.BlockSpec((_BR, _DIM), lambda i: (i, 0)),
        out_shape=jax.ShapeDtypeStruct((_VOCAB, _DIM), jnp.float32),
    )(w0, w1, w2, w3, s0, s1, s2, s3, scales)


def _sc_gather(table, idx):
    B = idx.shape[0]
    NW = 32
    b_per_w = B // NW
    C = 800
    n_chunks = b_per_w // C
    mesh = plsc.VectorSubcoreMesh(core_axis_name="c", subcore_axis_name="s")

    @functools.partial(
        pl.kernel,
        mesh=mesh,
        out_type=jax.ShapeDtypeStruct((B, _DIM), jnp.float32),
        scratch_types=[
            pltpu.VMEM((C,), jnp.int32),
            pltpu.VMEM((C, _DIM), jnp.float32),
            pltpu.SemaphoreType.DMA,
        ],
    )
    def k(table_hbm, idx_hbm, out_hbm, idx_v, rows_v, sem):
        wid = lax.axis_index("s") * 2 + lax.axis_index("c")
        base = wid * b_per_w

        def body(c, carry):
            off = base + c * C
            pltpu.sync_copy(idx_hbm.at[pl.ds(off, C)], idx_v)
            pltpu.async_copy(table_hbm.at[idx_v], rows_v, sem).wait()
            pltpu.sync_copy(rows_v, out_hbm.at[pl.ds(off, C)])
            return carry

        lax.fori_loop(0, n_chunks, body, 0)

    return k(table, idx)


def kernel(input_ids, w0, w1, w2, w3, s0, s1, s2, s3):
    scales = _compute_scales(w1, w2, w3, s1, s2, s3)
    table = _build_table(w0, w1, w2, w3, s0, s1, s2, s3, scales)
    idx = input_ids.reshape(-1).astype(jnp.int32)
    out = _sc_gather(table, idx)
    return out.reshape(input_ids.shape + (_DIM,))


# speedup vs baseline: 2.2973x; 2.2973x over previous
import functools

import jax
import jax.numpy as jnp
from jax import lax
from jax.experimental import pallas as pl
from jax.experimental.pallas import tpu as pltpu
from jax.experimental.pallas import tpu_sc as plsc

_DIM = 64
_VOCAB = 100000
_T1_ROWS = 1792
_T2_ROWS = 14336
_T3_ROWS = 83616
_W3_BLK = 6432
_BR = 256
_MV1, _MV2, _MV3 = 127.0, 31.0, 7.0


def _scales_body(w1, w2, w3, s1, s2, s3, out, acc):
    i = pl.program_id(0)

    @pl.when(i == 0)
    def _():
        acc[0:1, 0:_DIM] = jnp.zeros((1, _DIM), jnp.float32)
        acc[1:2, 0:_DIM] = jnp.max(jnp.abs(w1[...] * s1[...]), axis=0)[None, :]
        acc[2:3, 0:_DIM] = jnp.max(jnp.abs(w2[...] * s2[...]), axis=0)[None, :]

    m3 = jnp.max(jnp.abs(w3[...] * s3[...]), axis=0)[None, :]
    acc[0:1, 0:_DIM] = jnp.maximum(acc[0:1, 0:_DIM], m3)

    @pl.when(i == pl.num_programs(0) - 1)
    def _():
        sc1 = jnp.clip(jnp.max(acc[1:2, 0:_DIM]), 1e-8, None) / _MV1
        sc2 = jnp.clip(jnp.max(acc[2:3, 0:_DIM]), 1e-8, None) / _MV2
        sc3 = jnp.clip(jnp.max(acc[0:1, 0:_DIM]), 1e-8, None) / _MV3
        r = lax.broadcasted_iota(jnp.int32, (8, 128), 0)
        out[...] = jnp.where(r == 0, sc1, jnp.where(r == 1, sc2, sc3))


def _compute_scales(w1, w2, w3, s1, s2, s3):
    return pl.pallas_call(
        _scales_body,
        grid=(_T3_ROWS // _W3_BLK,),
        in_specs=[
            pl.BlockSpec((_T1_ROWS, _DIM), lambda i: (0, 0)),
            pl.BlockSpec((_T2_ROWS, _DIM), lambda i: (0, 0)),
            pl.BlockSpec((_W3_BLK, _DIM), lambda i: (i, 0)),
            pl.BlockSpec((1, _DIM), lambda i: (0, 0)),
            pl.BlockSpec((1, _DIM), lambda i: (0, 0)),
            pl.BlockSpec((1, _DIM), lambda i: (0, 0)),
        ],
        out_specs=pl.BlockSpec((8, 128), lambda i: (0, 0)),
        out_shape=jax.ShapeDtypeStruct((8, 128), jnp.float32),
        scratch_shapes=[pltpu.VMEM((8, 128), jnp.float32)],
    )(w1, w2, w3, s1, s2, s3)


def _quant_body(w0, w1, w2, w3, s0, s1, s2, s3, scales, out):
    i = pl.program_id(0)
    sc1 = scales[0, 0]
    sc2 = scales[1, 0]
    sc3 = scales[2, 0]

    def q(x, sc, mv):
        return jnp.clip(jnp.round(x / sc), -mv, mv) * sc

    x0 = w0[...] * s0[...]
    x1 = q(w1[...] * s1[...], sc1, _MV1)
    x2 = q(w2[...] * s2[...], sc2, _MV2)
    x3 = q(w3[...] * s3[...], sc3, _MV3)
    out[...] = jnp.where(i == 0, x0,
               jnp.where(i <= 7, x1,
               jnp.where(i <= 63, x2, x3)))


def _build_table(w0, w1, w2, w3, s0, s1, s2, s3, scales):
    n_blocks = pl.cdiv(_VOCAB, _BR)
    return pl.pallas_call(
        _quant_body,
        grid=(n_blocks,),
        in_specs=[
            pl.BlockSpec((_BR, _DIM), lambda i: (0, 0)),
            pl.BlockSpec((_BR, _DIM), lambda i: (jnp.clip(i - 1, 0, 6), 0)),
            pl.BlockSpec((_BR, _DIM), lambda i: (jnp.clip(i - 8, 0, 55), 0)),
            pl.BlockSpec((_BR, _DIM), lambda i: (jnp.clip(i - 64, 0, 326), 0)),
            pl.BlockSpec((1, _DIM), lambda i: (0, 0)),
            pl.BlockSpec((1, _DIM), lambda i: (0, 0)),
            pl.BlockSpec((1, _DIM), lambda i: (0, 0)),
            pl.BlockSpec((1, _DIM), lambda i: (0, 0)),
            pl.BlockSpec(memory_space=pltpu.SMEM),
        ],
        out_specs=pl.BlockSpec((_BR, _DIM), lambda i: (i, 0)),
        out_shape=jax.ShapeDtypeStruct((_VOCAB, _DIM), jnp.float32),
    )(w0, w1, w2, w3, s0, s1, s2, s3, scales)


def _sc_gather(table, idx):
    B = idx.shape[0]
    NW = 32
    b_per_w = B // NW
    C = 800
    n_chunks = b_per_w // C
    mesh = plsc.VectorSubcoreMesh(core_axis_name="c", subcore_axis_name="s")

    @functools.partial(
        pl.kernel,
        mesh=mesh,
        compiler_params=pltpu.CompilerParams(use_tc_tiling_on_sc=False),
        out_type=jax.ShapeDtypeStruct((B, _DIM), jnp.float32),
        scratch_types=[
            pltpu.VMEM((C,), jnp.int32),
            pltpu.VMEM((C, _DIM), jnp.float32),
            pltpu.SemaphoreType.DMA,
        ],
    )
    def k(table_hbm, idx_hbm, out_hbm, idx_v, rows_v, sem):
        wid = lax.axis_index("s") * 2 + lax.axis_index("c")
        base = wid * b_per_w

        def body(c, carry):
            off = base + c * C
            pltpu.sync_copy(idx_hbm.at[pl.ds(off, C)], idx_v)
            pltpu.async_copy(table_hbm.at[idx_v], rows_v, sem).wait()
            pltpu.sync_copy(rows_v, out_hbm.at[pl.ds(off, C)])
            return carry

        lax.fori_loop(0, n_chunks, body, 0)

    return k(table, idx)


def kernel(input_ids, w0, w1, w2, w3, s0, s1, s2, s3):
    scales = _compute_scales(w1, w2, w3, s1, s2, s3)
    table = _build_table(w0, w1, w2, w3, s0, s1, s2, s3, scales)
    idx = input_ids.reshape(-1).astype(jnp.int32)
    out = _sc_gather(table, idx)
    return out.reshape(input_ids.shape + (_DIM,))


# speedup vs baseline: 3.4320x; 1.4939x over previous
import functools

import jax
import jax.numpy as jnp
from jax import lax
from jax.experimental import pallas as pl
from jax.experimental.pallas import tpu as pltpu
from jax.experimental.pallas import tpu_sc as plsc

_DIM = 64
_VOCAB = 100000
_T1_ROWS = 1792
_T2_ROWS = 14336
_T3_ROWS = 83616
_W3_BLK = 6432
_BR = 2048
_MV1, _MV2, _MV3 = 127.0, 31.0, 7.0


def _scales_body(w1, w2, w3, s1, s2, s3, out, acc):
    i = pl.program_id(0)

    @pl.when(i == 0)
    def _():
        acc[0:1, 0:_DIM] = jnp.zeros((1, _DIM), jnp.float32)
        acc[1:2, 0:_DIM] = jnp.max(jnp.abs(w1[...] * s1[...]), axis=0)[None, :]
        acc[2:3, 0:_DIM] = jnp.max(jnp.abs(w2[...] * s2[...]), axis=0)[None, :]

    m3 = jnp.max(jnp.abs(w3[...] * s3[...]), axis=0)[None, :]
    acc[0:1, 0:_DIM] = jnp.maximum(acc[0:1, 0:_DIM], m3)

    @pl.when(i == pl.num_programs(0) - 1)
    def _():
        sc1 = jnp.clip(jnp.max(acc[1:2, 0:_DIM]), 1e-8, None) / _MV1
        sc2 = jnp.clip(jnp.max(acc[2:3, 0:_DIM]), 1e-8, None) / _MV2
        sc3 = jnp.clip(jnp.max(acc[0:1, 0:_DIM]), 1e-8, None) / _MV3
        r = lax.broadcasted_iota(jnp.int32, (8, 128), 0)
        out[...] = jnp.where(r == 0, sc1, jnp.where(r == 1, sc2, sc3))


def _compute_scales(w1, w2, w3, s1, s2, s3):
    return pl.pallas_call(
        _scales_body,
        grid=(_T3_ROWS // _W3_BLK,),
        in_specs=[
            pl.BlockSpec((_T1_ROWS, _DIM), lambda i: (0, 0)),
            pl.BlockSpec((_T2_ROWS, _DIM), lambda i: (0, 0)),
            pl.BlockSpec((_W3_BLK, _DIM), lambda i: (i, 0)),
            pl.BlockSpec((1, _DIM), lambda i: (0, 0)),
            pl.BlockSpec((1, _DIM), lambda i: (0, 0)),
            pl.BlockSpec((1, _DIM), lambda i: (0, 0)),
        ],
        out_specs=pl.BlockSpec((8, 128), lambda i: (0, 0)),
        out_shape=jax.ShapeDtypeStruct((8, 128), jnp.float32),
        scratch_shapes=[pltpu.VMEM((8, 128), jnp.float32)],
    )(w1, w2, w3, s1, s2, s3)


def _quant_body(w0, w1, w2, w3, s0, s1, s2, s3, scales, out):
    i = pl.program_id(0)

    def q(x, sc, mv):
        return jnp.clip(jnp.round(x * (1.0 / sc)), -mv, mv) * sc

    @pl.when(i == 0)
    def _():
        out[0:256, :] = w0[...] * s0[...]
        out[256:_BR, :] = q(w1[...] * s1[...], scales[0, 0], _MV1)

    @pl.when((i >= 1) & (i <= 7))
    def _():
        out[...] = q(w2[...] * s2[...], scales[1, 0], _MV2)

    @pl.when(i >= 8)
    def _():
        out[...] = q(w3[...] * s3[...], scales[2, 0], _MV3)


def _build_table(w0, w1, w2, w3, s0, s1, s2, s3, scales):
    n_blocks = pl.cdiv(_VOCAB, _BR)
    return pl.pallas_call(
        _quant_body,
        grid=(n_blocks,),
        in_specs=[
            pl.BlockSpec((256, _DIM), lambda i: (0, 0)),
            pl.BlockSpec((_T1_ROWS, _DIM), lambda i: (0, 0)),
            pl.BlockSpec((_BR, _DIM), lambda i: (jnp.clip(i - 1, 0, 6), 0)),
            pl.BlockSpec((_BR, _DIM), lambda i: (jnp.clip(i - 8, 0, 40), 0)),
            pl.BlockSpec((1, _DIM), lambda i: (0, 0)),
            pl.BlockSpec((1, _DIM), lambda i: (0, 0)),
            pl.BlockSpec((1, _DIM), lambda i: (0, 0)),
            pl.BlockSpec((1, _DIM), lambda i: (0, 0)),
            pl.BlockSpec(memory_space=pltpu.SMEM),
        ],
        out_specs=pl.BlockSpec((_BR, _DIM), lambda i: (i, 0)),
        out_shape=jax.ShapeDtypeStruct((_VOCAB, _DIM), jnp.float32),
    )(w0, w1, w2, w3, s0, s1, s2, s3, scales)


def _sc_gather(table, idx):
    B = idx.shape[0]
    NW = 32
    b_per_w = B // NW
    C = 800
    n_chunks = b_per_w // C
    mesh = plsc.VectorSubcoreMesh(core_axis_name="c", subcore_axis_name="s")

    @functools.partial(
        pl.kernel,
        mesh=mesh,
        compiler_params=pltpu.CompilerParams(use_tc_tiling_on_sc=False),
        out_type=jax.ShapeDtypeStruct((B, _DIM), jnp.float32),
        scratch_types=[
            pltpu.VMEM((n_chunks, C), jnp.int32),
            pltpu.VMEM((C, _DIM), jnp.float32),
            pltpu.VMEM((C, _DIM), jnp.float32),
            pltpu.SemaphoreType.DMA,
            pltpu.SemaphoreType.DMA,
            pltpu.SemaphoreType.DMA,
        ],
    )
    def k(table_hbm, idx_hbm, out_hbm, idx_v, rows0, rows1, sem_i, sem_g, sem_s):
        wid = lax.axis_index("s") * 2 + lax.axis_index("c")
        base = wid * b_per_w
        rows = (rows0, rows1)

        ih = [pltpu.async_copy(idx_hbm.at[pl.ds(base + c * C, C)],
                               idx_v.at[c], sem_i)
              for c in range(n_chunks)]
        for h in ih:
            h.wait()

        gh = [None] * n_chunks
        sh = [None] * n_chunks
        gh[0] = pltpu.async_copy(table_hbm.at[idx_v.at[0]], rows[0], sem_g)
        for c in range(n_chunks):
            if c + 1 < n_chunks:
                if c >= 1:
                    sh[c - 1].wait()
                gh[c + 1] = pltpu.async_copy(
                    table_hbm.at[idx_v.at[c + 1]], rows[(c + 1) % 2], sem_g)
            gh[c].wait()
            sh[c] = pltpu.async_copy(
                rows[c % 2], out_hbm.at[pl.ds(base + c * C, C)], sem_s)
        sh[n_chunks - 2].wait()
        sh[n_chunks - 1].wait()

    return k(table, idx)


def kernel(input_ids, w0, w1, w2, w3, s0, s1, s2, s3):
    scales = _compute_scales(w1, w2, w3, s1, s2, s3)
    table = _build_table(w0, w1, w2, w3, s0, s1, s2, s3, scales)
    idx = input_ids.reshape(-1).astype(jnp.int32)
    out = _sc_gather(table, idx)
    return out.reshape(input_ids.shape + (_DIM,))


# speedup vs baseline: 3.8927x; 1.1342x over previous
import functools

import jax
import jax.numpy as jnp
from jax import lax
from jax.experimental import pallas as pl
from jax.experimental.pallas import tpu as pltpu
from jax.experimental.pallas import tpu_sc as plsc

_DIM = 64
_VOCAB = 100000
_NW = 32
_MAGIC = 12582912.0
_MV = (127.0, 31.0, 7.0)

_N0 = 256 * _DIM // _NW
_N1 = 1792 * _DIM // _NW
_N2 = 14336 * _DIM // _NW
_N3 = 83616 * _DIM // _NW
_C3 = _N3 // 3

_O0 = 0
_O1 = 256 * _DIM
_O2 = 2048 * _DIM
_O3 = 16384 * _DIM

_mesh = functools.partial(
    pl.kernel,
    mesh=plsc.VectorSubcoreMesh(core_axis_name="c", subcore_axis_name="s"),
    compiler_params=pltpu.CompilerParams(use_tc_tiling_on_sc=False),
)


def _tile_id():
    return lax.axis_index("s") * 2 + lax.axis_index("c")


def _reduce_chunk(buf, nelem, unroll, carry):
    nv = nelem // 16
    iters = nv // unroll
    assert iters * unroll == nv and unroll % 4 == 0

    def body(it, ms):
        ms = list(ms)
        base = it * unroll
        for j in range(unroll):
            v = buf[pl.ds((base + j) * 16, 16)]
            ms[j % 4] = jnp.maximum(ms[j % 4], jnp.abs(v))
        return tuple(ms)

    return lax.fori_loop(0, iters, body, carry)


def _sc_scales(w1f, w2f, w3f):
    @functools.partial(
        _mesh,
        out_type=jax.ShapeDtypeStruct((_NW, 12, 16), jnp.float32),
        scratch_types=[
            pltpu.VMEM((_C3,), jnp.float32),
            pltpu.VMEM((12, 16), jnp.float32),
        ],
    )
    def k(w1_hbm, w2_hbm, w3_hbm, out_hbm, buf, acc):
        tid = _tile_id()
        zero4 = (jnp.zeros(16, jnp.float32),) * 4

        pltpu.sync_copy(w1_hbm.at[pl.ds(tid * _N1, _N1)], buf.at[pl.ds(0, _N1)])
        m1 = _reduce_chunk(buf, _N1, 28, zero4)

        pltpu.sync_copy(w2_hbm.at[pl.ds(tid * _N2, _N2)], buf.at[pl.ds(0, _N2)])
        m2 = _reduce_chunk(buf, _N2, 28, zero4)

        m3 = zero4
        for kk in range(3):
            pltpu.sync_copy(w3_hbm.at[pl.ds(tid * _N3 + kk * _C3, _C3)], buf)
            m3 = _reduce_chunk(buf, _C3, 52, m3)

        for c in range(4):
            acc[0 + c, :] = m1[c]
            acc[4 + c, :] = m2[c]
            acc[8 + c, :] = m3[c]
        pltpu.sync_copy(acc, out_hbm.at[tid])

    return k(w1f, w2f, w3f)


def _quant_chunk(buf, nelem, unroll, svecs, inv, sc):
    nv = nelem // 16
    iters = nv // unroll
    assert iters * unroll == nv and unroll % 4 == 0

    def body(it, carry):
        base = it * unroll
        for j in range(unroll):
            o = (base + j) * 16
            v = buf[pl.ds(o, 16)]
            y = (v * svecs[j % 4]) * inv
            r = (y + _MAGIC) - _MAGIC
            buf[pl.ds(o, 16)] = r * sc
        return carry

    lax.fori_loop(0, iters, body, 0)


def _sc_table(partials, w0f, w1f, w2f, w3f, s0f, s1f, s2f, s3f):
    @functools.partial(
        _mesh,
        out_type=jax.ShapeDtypeStruct((_VOCAB * _DIM,), jnp.float32),
        scratch_types=[
            pltpu.VMEM((_C3,), jnp.float32),
            pltpu.VMEM((_NW, 12, 16), jnp.float32),
            pltpu.VMEM((4, 64), jnp.float32),
        ],
    )
    def k(p_hbm, w0, w1, w2, w3, s0, s1, s2, s3, out_hbm, buf, pbuf, sbuf):
        tid = _tile_id()
        pltpu.sync_copy(p_hbm, pbuf)
        for i, s in enumerate((s0, s1, s2, s3)):
            pltpu.sync_copy(s, sbuf.at[i])

        svecs = [[sbuf[t, pl.ds(c * 16, 16)] for c in range(4)]
                 for t in range(4)]
        scs = []
        for t in range(3):
            m = [pbuf[0, t * 4 + c, :] for c in range(4)]
            for i in range(1, _NW):
                for c in range(4):
                    m[c] = jnp.maximum(m[c], pbuf[i, t * 4 + c, :])
            mm = jnp.maximum(jnp.maximum(m[0] * jnp.abs(svecs[t + 1][0]),
                                         m[1] * jnp.abs(svecs[t + 1][1])),
                             jnp.maximum(m[2] * jnp.abs(svecs[t + 1][2]),
                                         m[3] * jnp.abs(svecs[t + 1][3])))
            mx = mm[0]
            for i in range(1, 16):
                mx = jnp.maximum(mx, mm[i])
            mb = jnp.zeros((16,), jnp.float32) + jnp.maximum(mx, 1e-8)
            sc = mb / _MV[t]
            inv = 1.0 / sc
            scs.append((inv, sc))

        pltpu.sync_copy(w0.at[pl.ds(tid * _N0, _N0)], buf.at[pl.ds(0, _N0)])

        def body0(it, carry):
            for j in range(4):
                o = (it * 4 + j) * 16
                buf[pl.ds(o, 16)] = buf[pl.ds(o, 16)] * svecs[0][j % 4]
            return carry

        lax.fori_loop(0, _N0 // 64, body0, 0)
        pltpu.sync_copy(buf.at[pl.ds(0, _N0)], out_hbm.at[pl.ds(_O0 + tid * _N0, _N0)])

        pltpu.sync_copy(w1.at[pl.ds(tid * _N1, _N1)], buf.at[pl.ds(0, _N1)])
        _quant_chunk(buf, _N1, 28, svecs[1], *scs[0])
        pltpu.sync_copy(buf.at[pl.ds(0, _N1)], out_hbm.at[pl.ds(_O1 + tid * _N1, _N1)])

        pltpu.sync_copy(w2.at[pl.ds(tid * _N2, _N2)], buf.at[pl.ds(0, _N2)])
        _quant_chunk(buf, _N2, 28, svecs[2], *scs[1])
        pltpu.sync_copy(buf.at[pl.ds(0, _N2)], out_hbm.at[pl.ds(_O2 + tid * _N2, _N2)])

        for kk in range(3):
            pltpu.sync_copy(w3.at[pl.ds(tid * _N3 + kk * _C3, _C3)], buf)
            _quant_chunk(buf, _C3, 52, svecs[3], *scs[2])
            pltpu.sync_copy(buf, out_hbm.at[pl.ds(_O3 + tid * _N3 + kk * _C3, _C3)])

    return k(partials, w0f, w1f, w2f, w3f, s0f, s1f, s2f, s3f)


def _sc_gather(table, idx):
    B = idx.shape[0]
    b_per_w = B // _NW
    C = 800
    n_chunks = b_per_w // C

    @functools.partial(
        _mesh,
        out_type=jax.ShapeDtypeStruct((B, _DIM), jnp.float32),
        scratch_types=[
            pltpu.VMEM((n_chunks, C), jnp.int32),
            pltpu.VMEM((C, _DIM), jnp.float32),
            pltpu.VMEM((C, _DIM), jnp.float32),
            pltpu.SemaphoreType.DMA,
            pltpu.SemaphoreType.DMA,
            pltpu.SemaphoreType.DMA,
        ],
    )
    def k(table_hbm, idx_hbm, out_hbm, idx_v, rows0, rows1, sem_i, sem_g, sem_s):
        base = _tile_id() * b_per_w
        rows = (rows0, rows1)

        ih = [pltpu.async_copy(idx_hbm.at[pl.ds(base + c * C, C)],
                               idx_v.at[c], sem_i)
              for c in range(n_chunks)]
        for h in ih:
            h.wait()

        gh = [None] * n_chunks
        sh = [None] * n_chunks
        gh[0] = pltpu.async_copy(table_hbm.at[idx_v.at[0]], rows[0], sem_g)
        for c in range(n_chunks):
            if c + 1 < n_chunks:
                if c >= 1:
                    sh[c - 1].wait()
                gh[c + 1] = pltpu.async_copy(
                    table_hbm.at[idx_v.at[c + 1]], rows[(c + 1) % 2], sem_g)
            gh[c].wait()
            sh[c] = pltpu.async_copy(
                rows[c % 2], out_hbm.at[pl.ds(base + c * C, C)], sem_s)
        sh[n_chunks - 2].wait()
        sh[n_chunks - 1].wait()

    return k(table, idx)


def kernel(input_ids, w0, w1, w2, w3, s0, s1, s2, s3):
    partials = _sc_scales(w1.reshape(-1), w2.reshape(-1), w3.reshape(-1))
    tablef = _sc_table(partials, w0.reshape(-1), w1.reshape(-1),
                       w2.reshape(-1), w3.reshape(-1),
                       s0.reshape(-1), s1.reshape(-1),
                       s2.reshape(-1), s3.reshape(-1))
    table = tablef.reshape(_VOCAB, _DIM)
    idx = input_ids.reshape(-1).astype(jnp.int32)
    out = _sc_gather(table, idx)
    return out.reshape(input_ids.shape + (_DIM,))
